# baseline (speedup 1.0000x reference)
import functools

import jax
import jax.numpy as jnp
from jax import lax
from jax.experimental import pallas as pl
from jax.experimental.pallas import tpu as pltpu
from jax.experimental.pallas import tpu_sc as plsc

_CHUNK = 128


def _gather_planar_sc(table, idx_t, B, L):
    V, D = table.shape
    N = B * L
    info = plsc.get_sparse_core_info()
    nw = info.num_cores * info.num_subcores
    n_tasks = N // _CHUNK
    t_per_w = n_tasks // nw
    assert t_per_w * nw == n_tasks and t_per_w % 2 == 0
    assert D % 8 == 0 and B % _CHUNK == 0
    db_n = D // 8
    bb_n = B // _CHUNK

    idx3 = idx_t.reshape(nw, t_per_w, _CHUNK)
    mesh = plsc.VectorSubcoreMesh(core_axis_name="c", subcore_axis_name="s")

    @functools.partial(
        pl.kernel,
        mesh=mesh,
        out_type=jax.ShapeDtypeStruct((N * D,), jnp.float32),
        compiler_params=pltpu.CompilerParams(use_tc_tiling_on_sc=False, needs_layout_passes=False),
        scratch_types=[
            pltpu.VMEM((t_per_w, _CHUNK), jnp.int32),
            pltpu.VMEM((_CHUNK, D), jnp.float32),
            pltpu.VMEM((_CHUNK, D), jnp.float32),
            pltpu.VMEM((D * _CHUNK,), jnp.float32),
            pltpu.VMEM((D * _CHUNK,), jnp.float32),
            pltpu.SemaphoreType.DMA,
            pltpu.SemaphoreType.DMA,
            pltpu.SemaphoreType.DMA,
            pltpu.SemaphoreType.DMA,
        ],
    )
    def k(table_hbm, idx_hbm, out_hbm, idx_v, rows0, rows1, rt0, rt1,
          gsem0, gsem1, osem0, osem1):
        wid = lax.axis_index("s") * info.num_cores + lax.axis_index("c")
        tbase = wid * t_per_w
        pltpu.sync_copy(idx_hbm.at[wid], idx_v)

        ridx = [lax.iota(jnp.int32, 16) + (cb * 16) for cb in range(8)]
        cidx = [jnp.full((16,), d, dtype=jnp.int32) for d in range(D)]

        def fire_gather(j, rows, gsem):
            pltpu.async_copy(table_hbm.at[idx_v.at[j]], rows, gsem)

        def wait_16k(buf2d, sem):
            pltpu.make_async_copy(table_hbm.at[pl.ds(0, _CHUNK)], buf2d, sem
                                  ).wait()

        def transpose_block(rows, rt):
            for d in range(D):
                for cb in range(8):
                    v = plsc.load_gather(rows, [ridx[cb], cidx[d]])
                    rt[pl.ds(d * _CHUNK + cb * 16, 16)] = v

        def fire_out(j, rt, osem):
            t = tbase + j
            l = t // bb_n
            bb = t - l * bb_n
            for db in range(db_n):
                off = (l * db_n * bb_n + db * bb_n + bb) * (8 * _CHUNK)
                pltpu.async_copy(
                    rt.at[pl.ds(db * 8 * _CHUNK, 8 * _CHUNK)],
                    out_hbm.at[pl.ds(off, 8 * _CHUNK)],
                    osem,
                )

        def drain_out(rt, osem):
            pltpu.make_async_copy(out_hbm.at[pl.ds(0, D * _CHUNK)], rt, osem
                                  ).wait()

        def slot(j, fire_next, wait_prev_out,
                 rows_a, gsem_a, rt_a, osem_a, rows_b, gsem_b):
            @pl.when(fire_next)
            def _():
                fire_gather(j + 1, rows_b, gsem_b)

            wait_16k(rows_a, gsem_a)

            @pl.when(wait_prev_out)
            def _():
                drain_out(rt_a, osem_a)

            transpose_block(rows_a, rt_a)
            fire_out(j, rt_a, osem_a)

        fire_gather(0, rows0, gsem0)

        def pair(i, carry):
            true_ = i >= 0
            slot(2 * i, true_, i >= 1,
                 rows0, gsem0, rt0, osem0, rows1, gsem1)
            slot(2 * i + 1, i < t_per_w // 2 - 1, i >= 1,
                 rows1, gsem1, rt1, osem1, rows0, gsem0)
            return carry

        lax.fori_loop(0, t_per_w // 2, pair, 0)
        drain_out(rt0, osem0)
        drain_out(rt1, osem1)

    return k(table, idx3)


def kernel(token_ids, embedding_weights):
    B, L = token_ids.shape
    V, D = embedding_weights.shape
    idx_t = token_ids.T.reshape(B * L)
    flat = _gather_planar_sc(embedding_weights, idx_t, B, L)
    l6 = flat.reshape(L, D // 8, B // 128, 8, 128)
    return l6.transpose(2, 4, 0, 1, 3).reshape(B, L, D)


# speedup vs baseline: 1.1907x; 1.1907x over previous
import functools

import jax
import jax.numpy as jnp
from jax import lax
from jax.experimental import pallas as pl
from jax.experimental.pallas import tpu as pltpu
from jax.experimental.pallas import tpu_sc as plsc

_CHUNK = 128


def _gather_planar_sc(table, idx_t, B, L):
    V, D = table.shape
    N = B * L
    info = plsc.get_sparse_core_info()
    nw = info.num_cores * info.num_subcores
    n_tasks = N // _CHUNK
    t_per_w = n_tasks // nw
    assert t_per_w * nw == n_tasks and t_per_w % 2 == 0
    assert D % 8 == 0 and B % _CHUNK == 0
    db_n = D // 8
    bb_n = B // _CHUNK

    idx3 = idx_t.reshape(nw, t_per_w, _CHUNK)
    mesh = plsc.VectorSubcoreMesh(core_axis_name="c", subcore_axis_name="s")

    @functools.partial(
        pl.kernel,
        mesh=mesh,
        out_type=jax.ShapeDtypeStruct((N * D,), jnp.float32),
        compiler_params=pltpu.CompilerParams(use_tc_tiling_on_sc=False, needs_layout_passes=False),
        scratch_types=[
            pltpu.VMEM((t_per_w, _CHUNK), jnp.int32),
            pltpu.VMEM((_CHUNK, D), jnp.float32),
            pltpu.VMEM((_CHUNK, D), jnp.float32),
            pltpu.VMEM((D * _CHUNK,), jnp.float32),
            pltpu.VMEM((D * _CHUNK,), jnp.float32),
            pltpu.SemaphoreType.DMA,
            pltpu.SemaphoreType.DMA,
            pltpu.SemaphoreType.DMA,
            pltpu.SemaphoreType.DMA,
        ],
    )
    def k(table_hbm, idx_hbm, out_hbm, idx_v, rows0, rows1, rt0, rt1,
          gsem0, gsem1, osem0, osem1):
        wid = lax.axis_index("s") * info.num_cores + lax.axis_index("c")
        tbase = wid * t_per_w
        pltpu.sync_copy(idx_hbm.at[wid], idx_v)

        ridx = [lax.iota(jnp.int32, 16) + (cb * 16) for cb in range(8)]
        cidx = [jnp.full((16,), d, dtype=jnp.int32) for d in range(D)]

        def fire_gather(j, rows, gsem):
            pltpu.async_copy(table_hbm.at[idx_v.at[j]], rows, gsem)

        def wait_16k(buf2d, sem):
            pltpu.make_async_copy(table_hbm.at[pl.ds(0, _CHUNK)], buf2d, sem
                                  ).wait()

        def transpose_block(rows, rt):
            for d in range(D):
                vs = [plsc.load_gather(rows, [ridx[cb], cidx[d]])
                      for cb in range(8)]
                for cb in range(8):
                    rt[pl.ds(d * _CHUNK + cb * 16, 16)] = vs[cb]

        def fire_out(j, rt, osem):
            t = tbase + j
            l = t // bb_n
            bb = t - l * bb_n
            for db in range(db_n):
                off = (l * db_n * bb_n + db * bb_n + bb) * (8 * _CHUNK)
                pltpu.async_copy(
                    rt.at[pl.ds(db * 8 * _CHUNK, 8 * _CHUNK)],
                    out_hbm.at[pl.ds(off, 8 * _CHUNK)],
                    osem,
                )

        def drain_out(rt, osem):
            pltpu.make_async_copy(out_hbm.at[pl.ds(0, D * _CHUNK)], rt, osem
                                  ).wait()

        def slot(j, fire_next, wait_prev_out,
                 rows_a, gsem_a, rt_a, osem_a, rows_b, gsem_b):
            @pl.when(fire_next)
            def _():
                fire_gather(j + 1, rows_b, gsem_b)

            wait_16k(rows_a, gsem_a)

            @pl.when(wait_prev_out)
            def _():
                drain_out(rt_a, osem_a)

            transpose_block(rows_a, rt_a)
            fire_out(j, rt_a, osem_a)

        fire_gather(0, rows0, gsem0)

        def pair(i, carry):
            true_ = i >= 0
            slot(2 * i, true_, i >= 1,
                 rows0, gsem0, rt0, osem0, rows1, gsem1)
            slot(2 * i + 1, i < t_per_w // 2 - 1, i >= 1,
                 rows1, gsem1, rt1, osem1, rows0, gsem0)
            return carry

        lax.fori_loop(0, t_per_w // 2, pair, 0)
        drain_out(rt0, osem0)
        drain_out(rt1, osem1)

    return k(table, idx3)


def kernel(token_ids, embedding_weights):
    B, L = token_ids.shape
    V, D = embedding_weights.shape
    idx_t = token_ids.T.reshape(B * L)
    flat = _gather_planar_sc(embedding_weights, idx_t, B, L)
    l6 = flat.reshape(L, D // 8, B // 128, 8, 128)
    return l6.transpose(2, 4, 0, 1, 3).reshape(B, L, D)
